# Initial kernel scaffold; baseline (speedup 1.0000x reference)
import jax
import jax.numpy as jnp
from jax import lax
from jax.experimental import pallas as pl
from jax.experimental.pallas import tpu as pltpu
from jax.experimental.pallas import tpu_sc as plsc

N_NODES = 10000
D = 128
N_EDGES = 320000

NC, NS = 2, 16
NW = NC * NS
CH = 128
CPW = 80
EPW = CPW * CH
E_PAD = NW * EPW
ACC_ROWS = 10240
SLAB = ACC_ROWS // NS
OUT_SLAB = N_NODES // NS
NBUF = 4
ROW_BLK = 1000

_mesh = plsc.VectorSubcoreMesh(core_axis_name="c", subcore_axis_name="s")


def _deg_body(dst_idx, zvec, out, dst_v, ones_v, acc, m0, m1, m2, m3):
    c = lax.axis_index("c")
    sid = lax.axis_index("s")
    w = c * NS + sid
    sems = (m0, m1, m2, m3)

    pltpu.sync_copy(zvec, acc.at[pl.ds(sid * SLAB, SLAB)])
    for i in range(CH // 16):
        ones_v[pl.ds(i * 16, 16)] = jnp.full((16,), 1.0, jnp.float32)
    plsc.subcore_barrier()

    pltpu.sync_copy(dst_idx.at[w], dst_v)

    for b in range(NBUF):
        pltpu.async_copy(ones_v, acc.at[dst_v.at[b]], sems[b], add=True)

    @pl.loop(0, CPW - NBUF, step=NBUF)
    def _steady(j0):
        for b in range(NBUF):
            j = j0 + b
            pltpu.make_async_copy(ones_v, acc.at[dst_v.at[j]], sems[b]).wait()
            pltpu.async_copy(ones_v, acc.at[dst_v.at[j + NBUF]], sems[b],
                             add=True)

    for b in range(NBUF):
        j = CPW - NBUF + b
        pltpu.make_async_copy(ones_v, acc.at[dst_v.at[j]], sems[b]).wait()

    plsc.subcore_barrier()
    pltpu.sync_copy(acc.at[pl.ds(sid * SLAB, SLAB)],
                    out.at[c, pl.ds(sid * SLAB, SLAB)])


_sc_degree = pl.kernel(
    _deg_body,
    out_type=jax.ShapeDtypeStruct((NC, ACC_ROWS), jnp.float32),
    mesh=_mesh,
    scratch_types=[
        pltpu.VMEM((CPW, CH), jnp.int32),
        pltpu.VMEM((CH,), jnp.float32),
        pltpu.VMEM_SHARED((ACC_ROWS,), jnp.float32),
        pltpu.SemaphoreType.DMA,
        pltpu.SemaphoreType.DMA,
        pltpu.SemaphoreType.DMA,
        pltpu.SemaphoreType.DMA,
    ],
)


def _agg_body(hp, src_idx, dst_idx, zrow, out,
              src_v, dst_v, b0, b1, b2, b3, acc,
              g0, g1, g2, g3, s0, s1, s2, s3):
    c = lax.axis_index("c")
    sid = lax.axis_index("s")
    w = c * NS + sid
    bufs = (b0, b1, b2, b3)
    gsem = (g0, g1, g2, g3)
    ssem = (s0, s1, s2, s3)

    pltpu.sync_copy(zrow, acc.at[pl.ds(sid * SLAB, SLAB)])
    plsc.subcore_barrier()

    pltpu.sync_copy(src_idx.at[w], src_v)
    pltpu.sync_copy(dst_idx.at[w], dst_v)

    for b in range(NBUF):
        pltpu.async_copy(hp.at[src_v.at[b]], bufs[b], gsem[b])

    @pl.loop(0, CPW - NBUF, step=NBUF)
    def _steady(j0):
        for b in range(NBUF):
            j = j0 + b
            pltpu.make_async_copy(hp.at[src_v.at[j]], bufs[b], gsem[b]).wait()
            pltpu.async_copy(bufs[b], acc.at[dst_v.at[j]], ssem[b],
                             add=True).wait()
            pltpu.async_copy(hp.at[src_v.at[j + NBUF]], bufs[b], gsem[b])

    for b in range(NBUF):
        j = CPW - NBUF + b
        pltpu.make_async_copy(hp.at[src_v.at[j]], bufs[b], gsem[b]).wait()
        pltpu.async_copy(bufs[b], acc.at[dst_v.at[j]], ssem[b], add=True).wait()

    plsc.subcore_barrier()
    pltpu.sync_copy(acc.at[pl.ds(sid * OUT_SLAB, OUT_SLAB)],
                    out.at[c, pl.ds(sid * OUT_SLAB, OUT_SLAB)])


_sc_agg = pl.kernel(
    _agg_body,
    out_type=jax.ShapeDtypeStruct((NC, N_NODES, D), jnp.float32),
    mesh=_mesh,
    scratch_types=[
        pltpu.VMEM((CPW, CH), jnp.int32),
        pltpu.VMEM((CPW, CH), jnp.int32),
        pltpu.VMEM((CH, D), jnp.float32),
        pltpu.VMEM((CH, D), jnp.float32),
        pltpu.VMEM((CH, D), jnp.float32),
        pltpu.VMEM((CH, D), jnp.float32),
        pltpu.VMEM_SHARED((ACC_ROWS, D), jnp.float32),
        pltpu.SemaphoreType.DMA,
        pltpu.SemaphoreType.DMA,
        pltpu.SemaphoreType.DMA,
        pltpu.SemaphoreType.DMA,
        pltpu.SemaphoreType.DMA,
        pltpu.SemaphoreType.DMA,
        pltpu.SemaphoreType.DMA,
        pltpu.SemaphoreType.DMA,
    ],
)


def _tc_in_body(x_ref, w_ref, d_ref, hp_ref, s_ref):
    deg = d_ref[0] + d_ref[1] + 1.0
    s = lax.rsqrt(deg)
    s_ref[...] = s
    h = jnp.dot(x_ref[...], w_ref[...], preferred_element_type=jnp.float32)
    hp_ref[...] = h * s


def _tc_in(x, W1, degp):
    return pl.pallas_call(
        _tc_in_body,
        grid=(N_NODES // ROW_BLK,),
        in_specs=[
            pl.BlockSpec((ROW_BLK, D), lambda i: (i, 0)),
            pl.BlockSpec((D, D), lambda i: (0, 0)),
            pl.BlockSpec((NC, ROW_BLK, 1), lambda i: (0, i, 0)),
        ],
        out_specs=[
            pl.BlockSpec((ROW_BLK, D), lambda i: (i, 0)),
            pl.BlockSpec((ROW_BLK, 1), lambda i: (i, 0)),
        ],
        out_shape=[
            jax.ShapeDtypeStruct((N_NODES, D), jnp.float32),
            jax.ShapeDtypeStruct((N_NODES, 1), jnp.float32),
        ],
    )(x, W1, degp)


def _tc_mid_body(p_ref, hp_ref, s_ref, b_ref, w_ref, o_ref):
    agg = p_ref[0] + p_ref[1] + hp_ref[...]
    s = s_ref[...]
    z = jnp.maximum(s * agg + b_ref[...], 0.0)
    o_ref[...] = jnp.dot(z, w_ref[...], preferred_element_type=jnp.float32) * s


def _tc_mid(p, hp, s, b1, W2):
    return pl.pallas_call(
        _tc_mid_body,
        grid=(N_NODES // ROW_BLK,),
        in_specs=[
            pl.BlockSpec((NC, ROW_BLK, D), lambda i: (0, i, 0)),
            pl.BlockSpec((ROW_BLK, D), lambda i: (i, 0)),
            pl.BlockSpec((ROW_BLK, 1), lambda i: (i, 0)),
            pl.BlockSpec((1, D), lambda i: (0, 0)),
            pl.BlockSpec((D, D), lambda i: (0, 0)),
        ],
        out_specs=pl.BlockSpec((ROW_BLK, D), lambda i: (i, 0)),
        out_shape=jax.ShapeDtypeStruct((N_NODES, D), jnp.float32),
    )(p, hp, s, b1, W2)


def _tc_out_body(q_ref, hp_ref, s_ref, b_ref, o_ref):
    agg = q_ref[0] + q_ref[1] + hp_ref[...]
    o_ref[...] = jnp.maximum(s_ref[...] * agg + b_ref[...], 0.0)


def _tc_out(q, hp, s, b2):
    return pl.pallas_call(
        _tc_out_body,
        grid=(N_NODES // ROW_BLK,),
        in_specs=[
            pl.BlockSpec((NC, ROW_BLK, D), lambda i: (0, i, 0)),
            pl.BlockSpec((ROW_BLK, D), lambda i: (i, 0)),
            pl.BlockSpec((ROW_BLK, 1), lambda i: (i, 0)),
            pl.BlockSpec((1, D), lambda i: (0, 0)),
        ],
        out_specs=pl.BlockSpec((ROW_BLK, D), lambda i: (i, 0)),
        out_shape=jax.ShapeDtypeStruct((N_NODES, D), jnp.float32),
    )(q, hp, s, b2)


def kernel(x, edge_index, W1, b1, W2, b2):
    src = edge_index[0].astype(jnp.int32)
    dst = edge_index[1].astype(jnp.int32)
    padn = E_PAD - N_EDGES
    src3 = jnp.concatenate(
        [src, jnp.zeros((padn,), jnp.int32)]).reshape(NW, CPW, CH)
    dst3 = jnp.concatenate(
        [dst, jnp.full((padn,), N_NODES, jnp.int32)]).reshape(NW, CPW, CH)
    zrow = jnp.zeros((SLAB, D), jnp.float32)
    zvec = jnp.zeros((SLAB,), jnp.float32)

    degp = _sc_degree(dst3, zvec)
    degp = degp[:, :N_NODES, None]
    h1p, s = _tc_in(x, W1, degp)
    p = _sc_agg(h1p, src3, dst3, zrow)
    h2p = _tc_mid(p, h1p, s, b1.reshape(1, D), W2)
    q = _sc_agg(h2p, src3, dst3, zrow)
    return _tc_out(q, h2p, s, b2.reshape(1, D))


# speedup vs baseline: 9.4548x; 9.4548x over previous
import jax
import jax.numpy as jnp
from jax import lax
from jax.experimental import pallas as pl
from jax.experimental.pallas import tpu as pltpu
from jax.experimental.pallas import tpu_sc as plsc

N_NODES = 10000
D = 128
N_EDGES = 320000

NC, NS = 2, 16
NW = NC * NS
CH = 128
CPW = 80
HCH = 40
EPW = CPW * CH
E_PAD = NW * EPW
ACC_ROWS = 10112
SLAB = ACC_ROWS // NS
DEG_ROWS = 10240
DEG_SLAB = DEG_ROWS // NS
NBUF = 2
ROW_BLK = 1000

_mesh = plsc.VectorSubcoreMesh(core_axis_name="c", subcore_axis_name="s")


def _deg_body(dst_idx, zvec, out, dst_v, ones_v, acc, m0, m1, m2, m3):
    c = lax.axis_index("c")
    sid = lax.axis_index("s")
    w = c * NS + sid
    sems = (m0, m1, m2, m3)
    npipe = 4

    pltpu.sync_copy(zvec, acc.at[pl.ds(sid * DEG_SLAB, DEG_SLAB)])
    for i in range(CH // 16):
        ones_v[pl.ds(i * 16, 16)] = jnp.full((16,), 1.0, jnp.float32)
    plsc.subcore_barrier()

    pltpu.sync_copy(dst_idx.at[w], dst_v)

    for b in range(npipe):
        pltpu.async_copy(ones_v, acc.at[dst_v.at[b]], sems[b], add=True)

    @pl.loop(0, CPW - npipe, step=npipe)
    def _steady(j0):
        for b in range(npipe):
            j = j0 + b
            pltpu.make_async_copy(ones_v, acc.at[dst_v.at[j]], sems[b]).wait()
            pltpu.async_copy(ones_v, acc.at[dst_v.at[j + npipe]], sems[b],
                             add=True)

    for b in range(npipe):
        j = CPW - npipe + b
        pltpu.make_async_copy(ones_v, acc.at[dst_v.at[j]], sems[b]).wait()

    plsc.subcore_barrier()
    pltpu.sync_copy(acc.at[pl.ds(sid * DEG_SLAB, DEG_SLAB)],
                    out.at[pl.ds(c * DEG_ROWS + sid * DEG_SLAB, DEG_SLAB)])


_sc_degree = pl.kernel(
    _deg_body,
    out_type=jax.ShapeDtypeStruct((NC * DEG_ROWS,), jnp.float32),
    mesh=_mesh,
    scratch_types=[
        pltpu.VMEM((CPW, CH), jnp.int32),
        pltpu.VMEM((CH,), jnp.float32),
        pltpu.VMEM_SHARED((DEG_ROWS,), jnp.float32),
        pltpu.SemaphoreType.DMA,
        pltpu.SemaphoreType.DMA,
        pltpu.SemaphoreType.DMA,
        pltpu.SemaphoreType.DMA,
    ],
)


def _agg_body(hp, src_idx, dst_idx, zrow, out,
              src_v, dst_v, b0, b1, acc,
              g0, g1, s0, s1):
    c = lax.axis_index("c")
    sid = lax.axis_index("s")
    w = c * NS + sid
    bufs = (b0, b1)
    gsem = (g0, g1)
    ssem = (s0, s1)

    pltpu.sync_copy(zrow, acc.at[pl.ds(sid * SLAB, SLAB)])
    plsc.subcore_barrier()

    for half in range(CPW // HCH):
        pltpu.sync_copy(src_idx.at[w, pl.ds(half * HCH, HCH)], src_v)
        pltpu.sync_copy(dst_idx.at[w, pl.ds(half * HCH, HCH)], dst_v)

        for b in range(NBUF):
            pltpu.async_copy(hp.at[src_v.at[b]], bufs[b], gsem[b])

        @pl.loop(0, HCH - NBUF, step=NBUF)
        def _steady(j0):
            for b in range(NBUF):
                j = j0 + b
                pltpu.make_async_copy(hp.at[src_v.at[j]], bufs[b],
                                      gsem[b]).wait()
                pltpu.async_copy(bufs[b], acc.at[dst_v.at[j]], ssem[b],
                                 add=True).wait()
                pltpu.async_copy(hp.at[src_v.at[j + NBUF]], bufs[b], gsem[b])

        for b in range(NBUF):
            j = HCH - NBUF + b
            pltpu.make_async_copy(hp.at[src_v.at[j]], bufs[b], gsem[b]).wait()
            pltpu.async_copy(bufs[b], acc.at[dst_v.at[j]], ssem[b],
                             add=True).wait()

    plsc.subcore_barrier()
    pltpu.sync_copy(acc.at[pl.ds(sid * SLAB, SLAB)],
                    out.at[c, pl.ds(sid * SLAB, SLAB)])


_sc_agg = pl.kernel(
    _agg_body,
    out_type=jax.ShapeDtypeStruct((NC, ACC_ROWS, D), jnp.float32),
    mesh=_mesh,
    scratch_types=[
        pltpu.VMEM((HCH, CH), jnp.int32),
        pltpu.VMEM((HCH, CH), jnp.int32),
        pltpu.VMEM((CH, D), jnp.float32),
        pltpu.VMEM((CH, D), jnp.float32),
        pltpu.VMEM_SHARED((ACC_ROWS, D), jnp.float32),
        pltpu.SemaphoreType.DMA,
        pltpu.SemaphoreType.DMA,
        pltpu.SemaphoreType.DMA,
        pltpu.SemaphoreType.DMA,
    ],
)


def _tc_in_body(x_ref, w_ref, d_ref, hp_ref, s_ref):
    deg = d_ref[0] + d_ref[1] + 1.0
    s = lax.rsqrt(deg)
    s_ref[...] = s
    h = jnp.dot(x_ref[...], w_ref[...], preferred_element_type=jnp.float32)
    hp_ref[...] = h * s


def _tc_in(x, W1, degp):
    return pl.pallas_call(
        _tc_in_body,
        grid=(N_NODES // ROW_BLK,),
        in_specs=[
            pl.BlockSpec((ROW_BLK, D), lambda i: (i, 0)),
            pl.BlockSpec((D, D), lambda i: (0, 0)),
            pl.BlockSpec((NC, ROW_BLK, 1), lambda i: (0, i, 0)),
        ],
        out_specs=[
            pl.BlockSpec((ROW_BLK, D), lambda i: (i, 0)),
            pl.BlockSpec((ROW_BLK, 1), lambda i: (i, 0)),
        ],
        out_shape=[
            jax.ShapeDtypeStruct((N_NODES, D), jnp.float32),
            jax.ShapeDtypeStruct((N_NODES, 1), jnp.float32),
        ],
    )(x, W1, degp)


def _tc_mid_body(p_ref, hp_ref, s_ref, b_ref, w_ref, o_ref):
    agg = p_ref[0] + p_ref[1] + hp_ref[...]
    s = s_ref[...]
    z = jnp.maximum(s * agg + b_ref[...], 0.0)
    o_ref[...] = jnp.dot(z, w_ref[...], preferred_element_type=jnp.float32) * s


def _tc_mid(p, hp, s, b1, W2):
    return pl.pallas_call(
        _tc_mid_body,
        grid=(N_NODES // ROW_BLK,),
        in_specs=[
            pl.BlockSpec((NC, ROW_BLK, D), lambda i: (0, i, 0)),
            pl.BlockSpec((ROW_BLK, D), lambda i: (i, 0)),
            pl.BlockSpec((ROW_BLK, 1), lambda i: (i, 0)),
            pl.BlockSpec((1, D), lambda i: (0, 0)),
            pl.BlockSpec((D, D), lambda i: (0, 0)),
        ],
        out_specs=pl.BlockSpec((ROW_BLK, D), lambda i: (i, 0)),
        out_shape=jax.ShapeDtypeStruct((N_NODES, D), jnp.float32),
    )(p, hp, s, b1, W2)


def _tc_out_body(q_ref, hp_ref, s_ref, b_ref, o_ref):
    agg = q_ref[0] + q_ref[1] + hp_ref[...]
    o_ref[...] = jnp.maximum(s_ref[...] * agg + b_ref[...], 0.0)


def _tc_out(q, hp, s, b2):
    return pl.pallas_call(
        _tc_out_body,
        grid=(N_NODES // ROW_BLK,),
        in_specs=[
            pl.BlockSpec((NC, ROW_BLK, D), lambda i: (0, i, 0)),
            pl.BlockSpec((ROW_BLK, D), lambda i: (i, 0)),
            pl.BlockSpec((ROW_BLK, 1), lambda i: (i, 0)),
            pl.BlockSpec((1, D), lambda i: (0, 0)),
        ],
        out_specs=pl.BlockSpec((ROW_BLK, D), lambda i: (i, 0)),
        out_shape=jax.ShapeDtypeStruct((N_NODES, D), jnp.float32),
    )(q, hp, s, b2)


def kernel(x, edge_index, W1, b1, W2, b2):
    src = edge_index[0].astype(jnp.int32)
    dst = edge_index[1].astype(jnp.int32)
    padn = E_PAD - N_EDGES
    src3 = jnp.concatenate(
        [src, jnp.zeros((padn,), jnp.int32)]).reshape(NW, CPW, CH)
    dst3 = jnp.concatenate(
        [dst, jnp.full((padn,), N_NODES, jnp.int32)]).reshape(NW, CPW, CH)
    zrow = jnp.zeros((SLAB, D), jnp.float32)
    zvec = jnp.zeros((DEG_SLAB,), jnp.float32)

    degp = _sc_degree(dst3, zvec).reshape(NC, DEG_ROWS)
    degp = degp[:, :N_NODES, None]
    h1p, s = _tc_in(x, W1, degp)
    p = _sc_agg(h1p, src3, dst3, zrow)
    h2p = _tc_mid(p, h1p, s, b1.reshape(1, D), W2)
    q = _sc_agg(h2p, src3, dst3, zrow)
    return _tc_out(q, h2p, s, b2.reshape(1, D))
